# baseline (speedup 1.0000x reference)
import functools

import jax
import jax.numpy as jnp
from jax import lax
from jax.experimental import pallas as pl
from jax.experimental.pallas import tpu as pltpu
from jax.experimental.pallas import tpu_sc as plsc

_N = 10000
_E = 160000
_DN = 128
_DE = 16
_A = 16
_CD = 32
_B = 64
_CONV_N = 3
_STEPS = 3

_NC, _NS = 2, 16
_NW = _NC * _NS
_CH = 128
_GRP = 1024
_EPW = 5 * _GRP
_EPAD = _NW * _EPW
_NGRP = _EPW // _GRP
_NROWP = 10112
_RPT = _NROWP // _NS

_f32 = jnp.float32


def _tc_node_embed(x, w, b):
    def body(x_ref, w_ref, b_ref, o_ref):
        o_ref[:_N, :] = jnp.maximum(
            jnp.dot(x_ref[...], w_ref[...], preferred_element_type=_f32)
            + b_ref[...], 0.0)
    return pl.pallas_call(
        body, out_shape=jax.ShapeDtypeStruct((_NROWP, _A), _f32))(x, w, b)


_BE = 4096


def _tc_we(ea, w_e1, b_e1, w_e2, b_e2):
    def body(ea_ref, w1_ref, b1_ref, w2_ref, b2_ref, o_ref):
        h2 = jnp.maximum(
            jnp.dot(ea_ref[...], w1_ref[...], preferred_element_type=_f32)
            + b1_ref[...], 0.0)
        o_ref[...] = (jnp.dot(h2, w2_ref[...], preferred_element_type=_f32)
                      + b2_ref[...])

    nb = _EPAD // _BE
    return pl.pallas_call(
        body, grid=(nb,),
        in_specs=[
            pl.BlockSpec((_BE, _DE), lambda i: (i, 0)),
            pl.BlockSpec((_DE, _CD), lambda i: (0, 0)),
            pl.BlockSpec((1, _CD), lambda i: (0, 0)),
            pl.BlockSpec((_CD, _A * _A), lambda i: (0, 0)),
            pl.BlockSpec((1, _A * _A), lambda i: (0, 0)),
        ],
        out_specs=pl.BlockSpec((_BE, _A * _A), lambda i: (i, 0)),
        out_shape=jax.ShapeDtypeStruct((_EPAD, _A * _A), _f32),
    )(ea, w_e1, b_e1, w_e2, b_e2)


def _tc_msg(xj, we):
    def body(xj_ref, we_ref, o_ref):
        xj_b = xj_ref[...]
        we_b = we_ref[...]
        acc = xj_b[:, 0][:, None] * we_b[:, 0:_A]
        for i in range(1, _A):
            acc = acc + xj_b[:, i][:, None] * we_b[:, i * _A:(i + 1) * _A]
        o_ref[...] = acc

    nb = _EPAD // _BE
    return pl.pallas_call(
        body, grid=(nb,),
        in_specs=[
            pl.BlockSpec((_BE, _A), lambda i: (i, 0)),
            pl.BlockSpec((_BE, _A * _A), lambda i: (i, 0)),
        ],
        out_specs=pl.BlockSpec((_BE, _A), lambda i: (i, 0)),
        out_shape=jax.ShapeDtypeStruct((_EPAD, _A), _f32),
    )(xj, we)


def _gru(m, h, wih, whh, bih, bhh):
    ir = jnp.dot(m, wih[0], preferred_element_type=_f32) + bih[0][None, :]
    iz = jnp.dot(m, wih[1], preferred_element_type=_f32) + bih[1][None, :]
    inn = jnp.dot(m, wih[2], preferred_element_type=_f32) + bih[2][None, :]
    hr = jnp.dot(h, whh[0], preferred_element_type=_f32) + bhh[0][None, :]
    hz = jnp.dot(h, whh[1], preferred_element_type=_f32) + bhh[1][None, :]
    hn = jnp.dot(h, whh[2], preferred_element_type=_f32) + bhh[2][None, :]
    rr = jax.nn.sigmoid(ir + hr)
    zz = jax.nn.sigmoid(iz + hz)
    nn_ = jnp.tanh(inn + rr * hn)
    return (1.0 - zz) * nn_ + zz * h


def _agg_m(acc_ref, deg_ref, cb_ref):
    s = acc_ref[0, :_N, :] + acc_ref[1, :_N, :]
    dg = deg_ref[0, :_N, :] + deg_ref[1, :_N, :]
    return jnp.maximum(s / jnp.maximum(dg, 1.0) + cb_ref[...], 0.0)


def _tc_update(acc, deg, h, conv_b, wih, whh, bih, bhh):
    def body(acc_ref, deg_ref, h_ref, cb_ref, wih_ref, whh_ref, bih_ref,
             bhh_ref, o_ref):
        m = _agg_m(acc_ref, deg_ref, cb_ref)
        o_ref[:_N, :] = _gru(m, h_ref[:_N, :], wih_ref, whh_ref, bih_ref,
                             bhh_ref)
    return pl.pallas_call(
        body, out_shape=jax.ShapeDtypeStruct((_NROWP, _A), _f32))(
            acc, deg, h, conv_b, wih, whh, bih, bhh)


def _tc_update_final(acc, deg, h, batch2d, conv_b, wih, whh, bih, bhh,
                     wiq, wir, wh4, bih4, wpq, wpr, bp):
    def body(acc_ref, deg_ref, h_ref, b_ref, cb_ref, wih_ref, whh_ref,
             bih_ref, bhh_ref, wiq_ref, wir_ref, wh_ref, bih4_ref,
             wpq_ref, wpr_ref, bp_ref, o_ref):
        m = _agg_m(acc_ref, deg_ref, cb_ref)
        xs = _gru(m, h_ref[:_N, :], wih_ref, whh_ref, bih_ref, bhh_ref)

        onehot = (b_ref[...] ==
                  lax.broadcasted_iota(jnp.int32, (_N, _B), 1)).astype(_f32)
        negmask = (onehot - 1.0) * 1e30

        q = jnp.zeros((_B, _A), _f32)
        r = jnp.zeros((_B, _A), _f32)
        h_s = jnp.zeros((_B, _A), _f32)
        c_s = jnp.zeros((_B, _A), _f32)
        for _ in range(_STEPS):
            g = [jnp.dot(q, wiq_ref[k], preferred_element_type=_f32)
                 + jnp.dot(r, wir_ref[k], preferred_element_type=_f32)
                 + jnp.dot(h_s, wh_ref[k], preferred_element_type=_f32)
                 + bih4_ref[k][None, :] for k in range(4)]
            c_s = jax.nn.sigmoid(g[1]) * c_s + jax.nn.sigmoid(g[0]) * jnp.tanh(g[2])
            h_s = jax.nn.sigmoid(g[3]) * jnp.tanh(c_s)
            q = h_s
            e_mat = lax.dot_general(xs, q, (((1,), (1,)), ((), ())),
                                    preferred_element_type=_f32)
            e_msk = e_mat + negmask
            emax = jnp.max(e_msk, axis=0, keepdims=True)
            a = jnp.exp(e_msk - emax) * onehot
            asum = jnp.sum(a, axis=0, keepdims=True)
            a = a / jnp.maximum(asum, 1e-16)
            r = lax.dot_general(a, xs, (((0,), (0,)), ((), ())),
                                preferred_element_type=_f32)
        o_ref[...] = (jnp.dot(q, wpq_ref[...], preferred_element_type=_f32)
                      + jnp.dot(r, wpr_ref[...], preferred_element_type=_f32)
                      + bp_ref[...])
    return pl.pallas_call(
        body, out_shape=jax.ShapeDtypeStruct((_B, 1), _f32))(
            acc, deg, h, batch2d, conv_b, wih, whh, bih, bhh,
            wiq, wir, wh4, bih4, wpq, wpr, bp)


def _sc_mesh():
    return plsc.VectorSubcoreMesh(core_axis_name="c", subcore_axis_name="s",
                                  num_cores=_NC, num_subcores=_NS)


_SC_PARAMS = pltpu.CompilerParams(use_tc_tiling_on_sc=False)


def _sc_gather(table, idx2d):
    @functools.partial(
        pl.kernel,
        out_type=jax.ShapeDtypeStruct((_EPAD, _A), _f32),
        mesh=_sc_mesh(),
        scratch_types=[
            pltpu.VMEM((_GRP // _CH, _CH), jnp.int32),
            pltpu.VMEM((_GRP, _A), _f32),
            pltpu.VMEM((_RPT, _A), _f32),
            pltpu.VMEM_SHARED((_NROWP, _A), _f32),
            pltpu.SemaphoreType.DMA,
        ],
        compiler_params=_SC_PARAMS,
    )
    def k(tab, idx, out, idxv, rows, stage, tab_sh, sem):
        sid = lax.axis_index("s")
        wid = sid * _NC + lax.axis_index("c")
        base = wid * _EPW

        trows = pl.ds(pl.multiple_of(sid * _RPT, 8), _RPT)
        pltpu.sync_copy(tab.at[trows, :], stage)
        pltpu.sync_copy(stage, tab_sh.at[trows, :])
        plsc.subcore_barrier()

        def grp(g, carry):
            off = pl.multiple_of(base + g * _GRP, _GRP)
            pltpu.sync_copy(
                idx.at[pl.ds(pl.multiple_of(off // _CH, 8), _GRP // _CH), :],
                idxv)
            descs = [
                pltpu.async_copy(tab_sh.at[idxv.at[j]],
                                 rows.at[pl.ds(j * _CH, _CH), :], sem)
                for j in range(_GRP // _CH)
            ]
            for d in descs:
                d.wait()
            pltpu.sync_copy(rows, out.at[pl.ds(off, _GRP), :])
            return carry

        lax.fori_loop(0, _NGRP, grp, 0)

    return k(table, idx2d)


def _zero_rows(buf, nrows):
    zv = jnp.zeros((_A,), _f32)

    def zb(i, c):
        buf[i, :] = zv
        return c

    lax.fori_loop(0, nrows, zb, 0)


def _sc_scatter(msg, idx2d, with_deg):
    acc_t = jax.ShapeDtypeStruct((_NC, _NROWP, _A), _f32)
    out_type = (acc_t, acc_t) if with_deg else acc_t
    scratch = [
        pltpu.VMEM((_GRP // _CH, _CH), jnp.int32),
        pltpu.VMEM((_GRP, _A), _f32),
        pltpu.VMEM((_RPT, _A), _f32),
        pltpu.VMEM_SHARED((_NROWP, _A), _f32),
    ]
    if with_deg:
        scratch += [
            pltpu.VMEM((_CH, _A), _f32),
            pltpu.VMEM_SHARED((_NROWP, _A), _f32),
        ]

    @functools.partial(pl.kernel, out_type=out_type, mesh=_sc_mesh(),
                       scratch_types=scratch, compiler_params=_SC_PARAMS)
    def k(msg_hbm, idx_hbm, *rest):
        if with_deg:
            acc_out, deg_out, idxv, mbuf, zbuf, acc_sh, ones, deg_sh = rest
        else:
            acc_out, idxv, mbuf, zbuf, acc_sh = rest
            deg_out = ones = deg_sh = None
        cid = lax.axis_index("c")
        sid = lax.axis_index("s")
        wid = sid * _NC + cid
        rows = pl.ds(pl.multiple_of(sid * _RPT, 8), _RPT)

        _zero_rows(zbuf, _RPT)
        pltpu.sync_copy(zbuf, acc_sh.at[rows, :])
        if with_deg:
            pltpu.sync_copy(zbuf, deg_sh.at[rows, :])
            ov = jnp.ones((_A,), _f32)

            def ob(i, c):
                ones[i, :] = ov
                return c

            lax.fori_loop(0, _CH, ob, 0)
        plsc.subcore_barrier()

        def grp(g, carry):
            off = pl.multiple_of(wid * _EPW + g * _GRP, _GRP)
            pltpu.sync_copy(
                idx_hbm.at[pl.ds(pl.multiple_of(off // _CH, 8),
                                 _GRP // _CH), :], idxv)
            pltpu.sync_copy(msg_hbm.at[pl.ds(off, _GRP), :], mbuf)
            for j in range(_GRP // _CH):
                pltpu.sync_copy(mbuf.at[pl.ds(j * _CH, _CH), :],
                                acc_sh.at[idxv.at[j]], add=True)
                if with_deg:
                    pltpu.sync_copy(ones, deg_sh.at[idxv.at[j]], add=True)
            return carry

        lax.fori_loop(0, _NGRP, grp, 0)
        plsc.subcore_barrier()

        pltpu.sync_copy(acc_sh.at[rows, :], zbuf)
        pltpu.sync_copy(zbuf, acc_out.at[cid, rows, :])
        if with_deg:
            pltpu.sync_copy(deg_sh.at[rows, :], zbuf)
            pltpu.sync_copy(zbuf, deg_out.at[cid, rows, :])

    return k(msg, idx2d)


def kernel(x, edge_index, edge_attr, batch, W_lin, b_lin, W_e1, b_e1, W_e2,
           b_e2, conv_bias, W_ih, W_hh, b_ih, b_hh, Wi, Wh, bi, bh, W_pred,
           b_pred):
    pad = _EPAD - _E
    src = jnp.concatenate([edge_index[0], jnp.zeros((pad,), jnp.int32)])
    dst = jnp.concatenate([edge_index[1],
                           jnp.full((pad,), _N, jnp.int32)])
    src2d = src.reshape(_EPAD // _CH, _CH)
    dst2d = dst.reshape(_EPAD // _CH, _CH)
    ea = jnp.concatenate([edge_attr, jnp.zeros((pad, _DE), _f32)])

    wih = jnp.stack([W_ih[k * _A:(k + 1) * _A].T for k in range(3)])
    whh = jnp.stack([W_hh[k * _A:(k + 1) * _A].T for k in range(3)])
    bih = b_ih.reshape(3, _A)
    bhh = b_hh.reshape(3, _A)
    wiq = jnp.stack([Wi[k * _A:(k + 1) * _A, :_A].T for k in range(4)])
    wir = jnp.stack([Wi[k * _A:(k + 1) * _A, _A:].T for k in range(4)])
    wh4 = jnp.stack([Wh[k * _A:(k + 1) * _A].T for k in range(4)])
    bih4 = bi.reshape(4, _A) + bh.reshape(4, _A)
    cb = conv_bias.reshape(1, _A)
    b_lin2 = b_lin.reshape(1, _A)
    b_e12 = b_e1.reshape(1, _CD)
    wpq, wpr = W_pred[:_A], W_pred[_A:]
    bp = b_pred.reshape(1, 1)
    batch2d = batch.reshape(_N, 1)

    h = _tc_node_embed(x, W_lin, b_lin2)
    we = _tc_we(ea, W_e1, b_e12, W_e2, b_e2.reshape(1, -1))
    deg = None
    for t in range(_CONV_N):
        xj = _sc_gather(h, src2d)
        msg = _tc_msg(xj, we)
        if t == 0:
            acc, deg = _sc_scatter(msg, dst2d, with_deg=True)
        else:
            acc = _sc_scatter(msg, dst2d, with_deg=False)
        if t < _CONV_N - 1:
            h = _tc_update(acc, deg, h, cb, wih, whh, bih, bhh)
        else:
            pred = _tc_update_final(acc, deg, h, batch2d, cb, wih, whh, bih,
                                    bhh, wiq, wir, wh4, bih4, wpq, wpr, bp)
    return pred.reshape(-1)


# speedup vs baseline: 2.6216x; 2.6216x over previous
import functools

import jax
import jax.numpy as jnp
from jax import lax
from jax.experimental import pallas as pl
from jax.experimental.pallas import tpu as pltpu
from jax.experimental.pallas import tpu_sc as plsc

_N = 10000
_E = 160000
_DN = 128
_DE = 16
_A = 16
_CD = 32
_B = 64
_CONV_N = 3
_STEPS = 3

_NC, _NS = 2, 16
_NW = _NC * _NS
_CH = 128
_GRP = 1024
_EPW = 5 * _GRP
_EPAD = _NW * _EPW
_NGRP = _EPW // _GRP
_NROWP = 10112
_RPT = _NROWP // _NS

_f32 = jnp.float32


def _tc_node_embed(x, w, b):
    def body(x_ref, w_ref, b_ref, o_ref):
        o_ref[:_N, :] = jnp.maximum(
            jnp.dot(x_ref[...], w_ref[...], preferred_element_type=_f32)
            + b_ref[...], 0.0)
    return pl.pallas_call(
        body, out_shape=jax.ShapeDtypeStruct((_NROWP, _A), _f32))(x, w, b)


_BE = 4096


def _tc_we(ea, w_e1, b_e1, w_e2, b_e2):
    def body(ea_ref, w1_ref, b1_ref, w2_ref, b2_ref, o_ref):
        h2 = jnp.maximum(
            jnp.dot(ea_ref[...], w1_ref[...], preferred_element_type=_f32)
            + b1_ref[...], 0.0)
        o_ref[...] = (jnp.dot(h2, w2_ref[...], preferred_element_type=_f32)
                      + b2_ref[...])

    nb = _EPAD // _BE
    return pl.pallas_call(
        body, grid=(nb,),
        in_specs=[
            pl.BlockSpec((_BE, _DE), lambda i: (i, 0)),
            pl.BlockSpec((_DE, _CD), lambda i: (0, 0)),
            pl.BlockSpec((1, _CD), lambda i: (0, 0)),
            pl.BlockSpec((_CD, _A * _A), lambda i: (0, 0)),
            pl.BlockSpec((1, _A * _A), lambda i: (0, 0)),
        ],
        out_specs=pl.BlockSpec((_BE, _A * _A), lambda i: (i, 0)),
        out_shape=jax.ShapeDtypeStruct((_EPAD, _A * _A), _f32),
    )(ea, w_e1, b_e1, w_e2, b_e2)


def _tc_msg(xj, we):
    def body(xj_ref, we_ref, o_ref):
        xj_b = xj_ref[...]
        we_b = we_ref[...]
        acc = xj_b[:, 0][:, None] * we_b[:, 0:_A]
        for i in range(1, _A):
            acc = acc + xj_b[:, i][:, None] * we_b[:, i * _A:(i + 1) * _A]
        o_ref[...] = acc

    nb = _EPAD // _BE
    return pl.pallas_call(
        body, grid=(nb,),
        in_specs=[
            pl.BlockSpec((_BE, _A), lambda i: (i, 0)),
            pl.BlockSpec((_BE, _A * _A), lambda i: (i, 0)),
        ],
        out_specs=pl.BlockSpec((_BE, _A), lambda i: (i, 0)),
        out_shape=jax.ShapeDtypeStruct((_EPAD, _A), _f32),
    )(xj, we)


def _gru(m, h, wih, whh, bih, bhh):
    ir = jnp.dot(m, wih[0], preferred_element_type=_f32) + bih[0][None, :]
    iz = jnp.dot(m, wih[1], preferred_element_type=_f32) + bih[1][None, :]
    inn = jnp.dot(m, wih[2], preferred_element_type=_f32) + bih[2][None, :]
    hr = jnp.dot(h, whh[0], preferred_element_type=_f32) + bhh[0][None, :]
    hz = jnp.dot(h, whh[1], preferred_element_type=_f32) + bhh[1][None, :]
    hn = jnp.dot(h, whh[2], preferred_element_type=_f32) + bhh[2][None, :]
    rr = jax.nn.sigmoid(ir + hr)
    zz = jax.nn.sigmoid(iz + hz)
    nn_ = jnp.tanh(inn + rr * hn)
    return (1.0 - zz) * nn_ + zz * h


def _agg_m(acc_ref, deg_ref, cb_ref):
    s = acc_ref[0, :_N, :] + acc_ref[1, :_N, :]
    dg = deg_ref[0, :_N, :] + deg_ref[1, :_N, :]
    return jnp.maximum(s / jnp.maximum(dg, 1.0) + cb_ref[...], 0.0)


def _tc_update(acc, deg, h, conv_b, wih, whh, bih, bhh):
    def body(acc_ref, deg_ref, h_ref, cb_ref, wih_ref, whh_ref, bih_ref,
             bhh_ref, o_ref):
        m = _agg_m(acc_ref, deg_ref, cb_ref)
        o_ref[:_N, :] = _gru(m, h_ref[:_N, :], wih_ref, whh_ref, bih_ref,
                             bhh_ref)
    return pl.pallas_call(
        body, out_shape=jax.ShapeDtypeStruct((_NROWP, _A), _f32))(
            acc, deg, h, conv_b, wih, whh, bih, bhh)


def _tc_update_final(acc, deg, h, batch2d, conv_b, wih, whh, bih, bhh,
                     wiq, wir, wh4, bih4, wpq, wpr, bp):
    def body(acc_ref, deg_ref, h_ref, b_ref, cb_ref, wih_ref, whh_ref,
             bih_ref, bhh_ref, wiq_ref, wir_ref, wh_ref, bih4_ref,
             wpq_ref, wpr_ref, bp_ref, o_ref):
        m = _agg_m(acc_ref, deg_ref, cb_ref)
        xs = _gru(m, h_ref[:_N, :], wih_ref, whh_ref, bih_ref, bhh_ref)

        onehot = (b_ref[...] ==
                  lax.broadcasted_iota(jnp.int32, (_N, _B), 1)).astype(_f32)
        negmask = (onehot - 1.0) * 1e30

        q = jnp.zeros((_B, _A), _f32)
        r = jnp.zeros((_B, _A), _f32)
        h_s = jnp.zeros((_B, _A), _f32)
        c_s = jnp.zeros((_B, _A), _f32)
        for _ in range(_STEPS):
            g = [jnp.dot(q, wiq_ref[k], preferred_element_type=_f32)
                 + jnp.dot(r, wir_ref[k], preferred_element_type=_f32)
                 + jnp.dot(h_s, wh_ref[k], preferred_element_type=_f32)
                 + bih4_ref[k][None, :] for k in range(4)]
            c_s = jax.nn.sigmoid(g[1]) * c_s + jax.nn.sigmoid(g[0]) * jnp.tanh(g[2])
            h_s = jax.nn.sigmoid(g[3]) * jnp.tanh(c_s)
            q = h_s
            e_mat = lax.dot_general(xs, q, (((1,), (1,)), ((), ())),
                                    preferred_element_type=_f32)
            e_msk = e_mat + negmask
            emax = jnp.max(e_msk, axis=0, keepdims=True)
            a = jnp.exp(e_msk - emax) * onehot
            asum = jnp.sum(a, axis=0, keepdims=True)
            a = a / jnp.maximum(asum, 1e-16)
            r = lax.dot_general(a, xs, (((0,), (0,)), ((), ())),
                                preferred_element_type=_f32)
        o_ref[...] = (jnp.dot(q, wpq_ref[...], preferred_element_type=_f32)
                      + jnp.dot(r, wpr_ref[...], preferred_element_type=_f32)
                      + bp_ref[...])
    return pl.pallas_call(
        body, out_shape=jax.ShapeDtypeStruct((_B, 1), _f32))(
            acc, deg, h, batch2d, conv_b, wih, whh, bih, bhh,
            wiq, wir, wh4, bih4, wpq, wpr, bp)


def _sc_mesh():
    return plsc.VectorSubcoreMesh(core_axis_name="c", subcore_axis_name="s",
                                  num_cores=_NC, num_subcores=_NS)


_SC_PARAMS = pltpu.CompilerParams(use_tc_tiling_on_sc=False)


def _sc_conv(table, src2d, dst2d, we, with_deg):
    acc_t = jax.ShapeDtypeStruct((_NC, _NROWP, _A), _f32)
    out_type = (acc_t, acc_t) if with_deg else acc_t
    scratch = [
        pltpu.VMEM((_GRP // _CH, _CH), jnp.int32),
        pltpu.VMEM((_GRP // _CH, _CH), jnp.int32),
        pltpu.VMEM((_CH, _A * _A), _f32),
        pltpu.VMEM((_CH, _A), _f32),
        pltpu.VMEM((_CH, _A), _f32),
        pltpu.VMEM((_RPT, _A), _f32),
        pltpu.VMEM_SHARED((_NROWP, _A), _f32),
        pltpu.VMEM_SHARED((_NROWP, _A), _f32),
        pltpu.SemaphoreType.DMA,
    ]
    if with_deg:
        scratch += [
            pltpu.VMEM((_CH, _A), _f32),
            pltpu.VMEM_SHARED((_NROWP, _A), _f32),
        ]

    @functools.partial(pl.kernel, out_type=out_type, mesh=_sc_mesh(),
                       scratch_types=scratch, compiler_params=_SC_PARAMS)
    def k(tab, src, dst, we_hbm, *rest):
        if with_deg:
            (acc_out, deg_out, srcv, dstv, webuf, xjbuf, mbuf, zbuf,
             tab_sh, acc_sh, sem, ones, deg_sh) = rest
        else:
            (acc_out, srcv, dstv, webuf, xjbuf, mbuf, zbuf,
             tab_sh, acc_sh, sem) = rest
            deg_out = ones = deg_sh = None
        cid = lax.axis_index("c")
        sid = lax.axis_index("s")
        wid = sid * _NC + cid
        rows = pl.ds(pl.multiple_of(sid * _RPT, 8), _RPT)

        pltpu.sync_copy(tab.at[rows, :], zbuf)
        pltpu.sync_copy(zbuf, tab_sh.at[rows, :])
        _zero_rows(zbuf, _RPT)
        pltpu.sync_copy(zbuf, acc_sh.at[rows, :])
        if with_deg:
            pltpu.sync_copy(zbuf, deg_sh.at[rows, :])
            ov = jnp.ones((_A,), _f32)

            def ob(i, c):
                ones[i, :] = ov
                return c

            lax.fori_loop(0, _CH, ob, 0)
        plsc.subcore_barrier()

        def grp(g, carry):
            off = pl.multiple_of(wid * _EPW + g * _GRP, _GRP)
            crow = pl.multiple_of(off // _CH, 8)
            pltpu.sync_copy(src.at[pl.ds(crow, _GRP // _CH), :], srcv)
            pltpu.sync_copy(dst.at[pl.ds(crow, _GRP // _CH), :], dstv)
            for j in range(_GRP // _CH):
                eoff = pl.multiple_of(off + j * _CH, _CH)
                pltpu.sync_copy(we_hbm.at[pl.ds(eoff, _CH), :], webuf)
                pltpu.async_copy(tab_sh.at[srcv.at[j]], xjbuf, sem).wait()

                def edge(e, c):
                    xvec = xjbuf[e, :]
                    acc = xvec[0] * webuf[e, pl.ds(0, _A)]
                    for i in range(1, _A):
                        acc = acc + xvec[i] * webuf[e, pl.ds(i * _A, _A)]
                    mbuf[e, :] = acc
                    return c

                lax.fori_loop(0, _CH, edge, 0, unroll=2)
                pltpu.sync_copy(mbuf, acc_sh.at[dstv.at[j]], add=True)
                if with_deg:
                    pltpu.sync_copy(ones, deg_sh.at[dstv.at[j]], add=True)
            return carry

        lax.fori_loop(0, _NGRP, grp, 0)
        plsc.subcore_barrier()

        pltpu.sync_copy(acc_sh.at[rows, :], zbuf)
        pltpu.sync_copy(zbuf, acc_out.at[cid, rows, :])
        if with_deg:
            pltpu.sync_copy(deg_sh.at[rows, :], zbuf)
            pltpu.sync_copy(zbuf, deg_out.at[cid, rows, :])

    return k(table, src2d, dst2d, we)


def _sc_gather(table, idx2d):
    @functools.partial(
        pl.kernel,
        out_type=jax.ShapeDtypeStruct((_EPAD, _A), _f32),
        mesh=_sc_mesh(),
        scratch_types=[
            pltpu.VMEM((_GRP // _CH, _CH), jnp.int32),
            pltpu.VMEM((_GRP, _A), _f32),
            pltpu.VMEM((_RPT, _A), _f32),
            pltpu.VMEM_SHARED((_NROWP, _A), _f32),
            pltpu.SemaphoreType.DMA,
        ],
        compiler_params=_SC_PARAMS,
    )
    def k(tab, idx, out, idxv, rows, stage, tab_sh, sem):
        sid = lax.axis_index("s")
        wid = sid * _NC + lax.axis_index("c")
        base = wid * _EPW

        trows = pl.ds(pl.multiple_of(sid * _RPT, 8), _RPT)
        pltpu.sync_copy(tab.at[trows, :], stage)
        pltpu.sync_copy(stage, tab_sh.at[trows, :])
        plsc.subcore_barrier()

        def grp(g, carry):
            off = pl.multiple_of(base + g * _GRP, _GRP)
            pltpu.sync_copy(
                idx.at[pl.ds(pl.multiple_of(off // _CH, 8), _GRP // _CH), :],
                idxv)
            descs = [
                pltpu.async_copy(tab_sh.at[idxv.at[j]],
                                 rows.at[pl.ds(j * _CH, _CH), :], sem)
                for j in range(_GRP // _CH)
            ]
            for d in descs:
                d.wait()
            pltpu.sync_copy(rows, out.at[pl.ds(off, _GRP), :])
            return carry

        lax.fori_loop(0, _NGRP, grp, 0)

    return k(table, idx2d)


def _zero_rows(buf, nrows):
    zv = jnp.zeros((_A,), _f32)

    def zb(i, c):
        buf[i, :] = zv
        return c

    lax.fori_loop(0, nrows, zb, 0)


def _sc_scatter(msg, idx2d, with_deg):
    acc_t = jax.ShapeDtypeStruct((_NC, _NROWP, _A), _f32)
    out_type = (acc_t, acc_t) if with_deg else acc_t
    scratch = [
        pltpu.VMEM((_GRP // _CH, _CH), jnp.int32),
        pltpu.VMEM((_GRP, _A), _f32),
        pltpu.VMEM((_RPT, _A), _f32),
        pltpu.VMEM_SHARED((_NROWP, _A), _f32),
    ]
    if with_deg:
        scratch += [
            pltpu.VMEM((_CH, _A), _f32),
            pltpu.VMEM_SHARED((_NROWP, _A), _f32),
        ]

    @functools.partial(pl.kernel, out_type=out_type, mesh=_sc_mesh(),
                       scratch_types=scratch, compiler_params=_SC_PARAMS)
    def k(msg_hbm, idx_hbm, *rest):
        if with_deg:
            acc_out, deg_out, idxv, mbuf, zbuf, acc_sh, ones, deg_sh = rest
        else:
            acc_out, idxv, mbuf, zbuf, acc_sh = rest
            deg_out = ones = deg_sh = None
        cid = lax.axis_index("c")
        sid = lax.axis_index("s")
        wid = sid * _NC + cid
        rows = pl.ds(pl.multiple_of(sid * _RPT, 8), _RPT)

        _zero_rows(zbuf, _RPT)
        pltpu.sync_copy(zbuf, acc_sh.at[rows, :])
        if with_deg:
            pltpu.sync_copy(zbuf, deg_sh.at[rows, :])
            ov = jnp.ones((_A,), _f32)

            def ob(i, c):
                ones[i, :] = ov
                return c

            lax.fori_loop(0, _CH, ob, 0)
        plsc.subcore_barrier()

        def grp(g, carry):
            off = pl.multiple_of(wid * _EPW + g * _GRP, _GRP)
            pltpu.sync_copy(
                idx_hbm.at[pl.ds(pl.multiple_of(off // _CH, 8),
                                 _GRP // _CH), :], idxv)
            pltpu.sync_copy(msg_hbm.at[pl.ds(off, _GRP), :], mbuf)
            for j in range(_GRP // _CH):
                pltpu.sync_copy(mbuf.at[pl.ds(j * _CH, _CH), :],
                                acc_sh.at[idxv.at[j]], add=True)
                if with_deg:
                    pltpu.sync_copy(ones, deg_sh.at[idxv.at[j]], add=True)
            return carry

        lax.fori_loop(0, _NGRP, grp, 0)
        plsc.subcore_barrier()

        pltpu.sync_copy(acc_sh.at[rows, :], zbuf)
        pltpu.sync_copy(zbuf, acc_out.at[cid, rows, :])
        if with_deg:
            pltpu.sync_copy(deg_sh.at[rows, :], zbuf)
            pltpu.sync_copy(zbuf, deg_out.at[cid, rows, :])

    return k(msg, idx2d)


def kernel(x, edge_index, edge_attr, batch, W_lin, b_lin, W_e1, b_e1, W_e2,
           b_e2, conv_bias, W_ih, W_hh, b_ih, b_hh, Wi, Wh, bi, bh, W_pred,
           b_pred):
    pad = _EPAD - _E
    src = jnp.concatenate([edge_index[0], jnp.zeros((pad,), jnp.int32)])
    dst = jnp.concatenate([edge_index[1],
                           jnp.full((pad,), _N, jnp.int32)])
    src2d = src.reshape(_EPAD // _CH, _CH)
    dst2d = dst.reshape(_EPAD // _CH, _CH)
    ea = jnp.concatenate([edge_attr, jnp.zeros((pad, _DE), _f32)])

    wih = jnp.stack([W_ih[k * _A:(k + 1) * _A].T for k in range(3)])
    whh = jnp.stack([W_hh[k * _A:(k + 1) * _A].T for k in range(3)])
    bih = b_ih.reshape(3, _A)
    bhh = b_hh.reshape(3, _A)
    wiq = jnp.stack([Wi[k * _A:(k + 1) * _A, :_A].T for k in range(4)])
    wir = jnp.stack([Wi[k * _A:(k + 1) * _A, _A:].T for k in range(4)])
    wh4 = jnp.stack([Wh[k * _A:(k + 1) * _A].T for k in range(4)])
    bih4 = bi.reshape(4, _A) + bh.reshape(4, _A)
    cb = conv_bias.reshape(1, _A)
    b_lin2 = b_lin.reshape(1, _A)
    b_e12 = b_e1.reshape(1, _CD)
    wpq, wpr = W_pred[:_A], W_pred[_A:]
    bp = b_pred.reshape(1, 1)
    batch2d = batch.reshape(_N, 1)

    h = _tc_node_embed(x, W_lin, b_lin2)
    we = _tc_we(ea, W_e1, b_e12, W_e2, b_e2.reshape(1, -1))
    deg = None
    for t in range(_CONV_N):
        if t == 0:
            acc, deg = _sc_conv(h, src2d, dst2d, we, with_deg=True)
        else:
            acc = _sc_conv(h, src2d, dst2d, we, with_deg=False)
        if t < _CONV_N - 1:
            h = _tc_update(acc, deg, h, cb, wih, whh, bih, bhh)
        else:
            pred = _tc_update_final(acc, deg, h, batch2d, cb, wih, whh, bih,
                                    bhh, wiq, wir, wh4, bih4, wpq, wpr, bp)
    return pred.reshape(-1)


# speedup vs baseline: 3.2873x; 1.2539x over previous
import functools

import jax
import jax.numpy as jnp
from jax import lax
from jax.experimental import pallas as pl
from jax.experimental.pallas import tpu as pltpu
from jax.experimental.pallas import tpu_sc as plsc

_N = 10000
_E = 160000
_DN = 128
_DE = 16
_A = 16
_CD = 32
_B = 64
_CONV_N = 3
_STEPS = 3

_NC, _NS = 2, 16
_NW = _NC * _NS
_CH = 128
_GRP = 1024
_EPW = 5 * _GRP
_EPAD = _NW * _EPW
_NGRP = _EPW // _GRP
_NROWP = 10112
_RPT = _NROWP // _NS

_f32 = jnp.float32


def _tc_node_embed(x, w, b):
    def body(x_ref, w_ref, b_ref, o_ref):
        o_ref[:_N, :] = jnp.maximum(
            jnp.dot(x_ref[...], w_ref[...], preferred_element_type=_f32)
            + b_ref[...], 0.0)
    return pl.pallas_call(
        body, out_shape=jax.ShapeDtypeStruct((_NROWP, _A), _f32))(x, w, b)


_BE = 4096


def _tc_we(ea, w_e1, b_e1, w_e2, b_e2):
    def body(ea_ref, w1_ref, b1_ref, w2_ref, b2_ref, o_ref):
        h2 = jnp.maximum(
            jnp.dot(ea_ref[...], w1_ref[...], preferred_element_type=_f32)
            + b1_ref[...], 0.0)
        o_ref[...] = (jnp.dot(h2, w2_ref[...], preferred_element_type=_f32)
                      + b2_ref[...])

    nb = _EPAD // _BE
    return pl.pallas_call(
        body, grid=(nb,),
        in_specs=[
            pl.BlockSpec((_BE, _DE), lambda i: (i, 0)),
            pl.BlockSpec((_DE, _CD), lambda i: (0, 0)),
            pl.BlockSpec((1, _CD), lambda i: (0, 0)),
            pl.BlockSpec((_CD, _A * _A), lambda i: (0, 0)),
            pl.BlockSpec((1, _A * _A), lambda i: (0, 0)),
        ],
        out_specs=pl.BlockSpec((_BE, _A * _A), lambda i: (i, 0)),
        out_shape=jax.ShapeDtypeStruct((_EPAD, _A * _A), _f32),
    )(ea, w_e1, b_e1, w_e2, b_e2)


def _tc_msg(xj, we):
    def body(xj_ref, we_ref, o_ref):
        xj_b = xj_ref[...]
        we_b = we_ref[...]
        acc = xj_b[:, 0][:, None] * we_b[:, 0:_A]
        for i in range(1, _A):
            acc = acc + xj_b[:, i][:, None] * we_b[:, i * _A:(i + 1) * _A]
        o_ref[...] = acc

    nb = _EPAD // _BE
    return pl.pallas_call(
        body, grid=(nb,),
        in_specs=[
            pl.BlockSpec((_BE, _A), lambda i: (i, 0)),
            pl.BlockSpec((_BE, _A * _A), lambda i: (i, 0)),
        ],
        out_specs=pl.BlockSpec((_BE, _A), lambda i: (i, 0)),
        out_shape=jax.ShapeDtypeStruct((_EPAD, _A), _f32),
    )(xj, we)


def _gru(m, h, wih, whh, bih, bhh):
    ir = jnp.dot(m, wih[0], preferred_element_type=_f32) + bih[0][None, :]
    iz = jnp.dot(m, wih[1], preferred_element_type=_f32) + bih[1][None, :]
    inn = jnp.dot(m, wih[2], preferred_element_type=_f32) + bih[2][None, :]
    hr = jnp.dot(h, whh[0], preferred_element_type=_f32) + bhh[0][None, :]
    hz = jnp.dot(h, whh[1], preferred_element_type=_f32) + bhh[1][None, :]
    hn = jnp.dot(h, whh[2], preferred_element_type=_f32) + bhh[2][None, :]
    rr = jax.nn.sigmoid(ir + hr)
    zz = jax.nn.sigmoid(iz + hz)
    nn_ = jnp.tanh(inn + rr * hn)
    return (1.0 - zz) * nn_ + zz * h


def _agg_m(acc_ref, deg_ref, cb_ref):
    s = acc_ref[0, :_N, :] + acc_ref[1, :_N, :]
    dg = deg_ref[0, :_N, :] + deg_ref[1, :_N, :]
    return jnp.maximum(s / jnp.maximum(dg, 1.0) + cb_ref[...], 0.0)


def _tc_update(acc, deg, h, conv_b, wih, whh, bih, bhh):
    def body(acc_ref, deg_ref, h_ref, cb_ref, wih_ref, whh_ref, bih_ref,
             bhh_ref, o_ref):
        m = _agg_m(acc_ref, deg_ref, cb_ref)
        o_ref[:_N, :] = _gru(m, h_ref[:_N, :], wih_ref, whh_ref, bih_ref,
                             bhh_ref)
    return pl.pallas_call(
        body, out_shape=jax.ShapeDtypeStruct((_NROWP, _A), _f32))(
            acc, deg, h, conv_b, wih, whh, bih, bhh)


def _tc_update_final(acc, deg, h, batch2d, conv_b, wih, whh, bih, bhh,
                     wiq, wir, wh4, bih4, wpq, wpr, bp):
    def body(acc_ref, deg_ref, h_ref, b_ref, cb_ref, wih_ref, whh_ref,
             bih_ref, bhh_ref, wiq_ref, wir_ref, wh_ref, bih4_ref,
             wpq_ref, wpr_ref, bp_ref, o_ref):
        m = _agg_m(acc_ref, deg_ref, cb_ref)
        xs = _gru(m, h_ref[:_N, :], wih_ref, whh_ref, bih_ref, bhh_ref)

        onehot = (b_ref[...] ==
                  lax.broadcasted_iota(jnp.int32, (_N, _B), 1)).astype(_f32)
        negmask = (onehot - 1.0) * 1e30

        q = jnp.zeros((_B, _A), _f32)
        r = jnp.zeros((_B, _A), _f32)
        h_s = jnp.zeros((_B, _A), _f32)
        c_s = jnp.zeros((_B, _A), _f32)
        for _ in range(_STEPS):
            g = [jnp.dot(q, wiq_ref[k], preferred_element_type=_f32)
                 + jnp.dot(r, wir_ref[k], preferred_element_type=_f32)
                 + jnp.dot(h_s, wh_ref[k], preferred_element_type=_f32)
                 + bih4_ref[k][None, :] for k in range(4)]
            c_s = jax.nn.sigmoid(g[1]) * c_s + jax.nn.sigmoid(g[0]) * jnp.tanh(g[2])
            h_s = jax.nn.sigmoid(g[3]) * jnp.tanh(c_s)
            q = h_s
            e_mat = lax.dot_general(xs, q, (((1,), (1,)), ((), ())),
                                    preferred_element_type=_f32)
            e_msk = e_mat + negmask
            emax = jnp.max(e_msk, axis=0, keepdims=True)
            a = jnp.exp(e_msk - emax) * onehot
            asum = jnp.sum(a, axis=0, keepdims=True)
            a = a / jnp.maximum(asum, 1e-16)
            r = lax.dot_general(a, xs, (((0,), (0,)), ((), ())),
                                preferred_element_type=_f32)
        o_ref[...] = (jnp.dot(q, wpq_ref[...], preferred_element_type=_f32)
                      + jnp.dot(r, wpr_ref[...], preferred_element_type=_f32)
                      + bp_ref[...])
    return pl.pallas_call(
        body, out_shape=jax.ShapeDtypeStruct((_B, 1), _f32))(
            acc, deg, h, batch2d, conv_b, wih, whh, bih, bhh,
            wiq, wir, wh4, bih4, wpq, wpr, bp)


def _sc_mesh():
    return plsc.VectorSubcoreMesh(core_axis_name="c", subcore_axis_name="s",
                                  num_cores=_NC, num_subcores=_NS)


_SC_PARAMS = pltpu.CompilerParams(use_tc_tiling_on_sc=False)


def _sc_conv(table, src2d, dst2d, we, with_deg):
    acc_t = jax.ShapeDtypeStruct((_NC, _NROWP, _A), _f32)
    out_type = (acc_t, acc_t) if with_deg else acc_t
    scratch = [
        pltpu.VMEM((_GRP // _CH, _CH), jnp.int32),
        pltpu.VMEM((_GRP // _CH, _CH), jnp.int32),
        pltpu.VMEM((_CH, _A * _A), _f32),
        pltpu.VMEM((_CH, _A * _A), _f32),
        pltpu.VMEM((_CH, _A), _f32),
        pltpu.VMEM((_CH, _A), _f32),
        pltpu.VMEM((_RPT, _A), _f32),
        pltpu.VMEM_SHARED((_NROWP, _A), _f32),
        pltpu.VMEM_SHARED((_NROWP, _A), _f32),
        pltpu.SemaphoreType.DMA,
        pltpu.SemaphoreType.DMA,
        pltpu.SemaphoreType.DMA,
    ]
    if with_deg:
        scratch += [
            pltpu.VMEM((_CH, _A), _f32),
            pltpu.VMEM_SHARED((_NROWP, _A), _f32),
        ]

    @functools.partial(pl.kernel, out_type=out_type, mesh=_sc_mesh(),
                       scratch_types=scratch, compiler_params=_SC_PARAMS)
    def k(tab, src, dst, we_hbm, *rest):
        if with_deg:
            (acc_out, deg_out, srcv, dstv, web0, web1, xjbuf, mbuf, zbuf,
             tab_sh, acc_sh, sem, wsem0, wsem1, ones, deg_sh) = rest
        else:
            (acc_out, srcv, dstv, web0, web1, xjbuf, mbuf, zbuf,
             tab_sh, acc_sh, sem, wsem0, wsem1) = rest
            deg_out = ones = deg_sh = None
        webuf = (web0, web1)
        wsem = (wsem0, wsem1)
        cid = lax.axis_index("c")
        sid = lax.axis_index("s")
        wid = sid * _NC + cid
        rows = pl.ds(pl.multiple_of(sid * _RPT, 8), _RPT)

        pltpu.sync_copy(tab.at[rows, :], zbuf)
        pltpu.sync_copy(zbuf, tab_sh.at[rows, :])
        _zero_rows(zbuf, _RPT)
        pltpu.sync_copy(zbuf, acc_sh.at[rows, :])
        if with_deg:
            pltpu.sync_copy(zbuf, deg_sh.at[rows, :])
            ov = jnp.ones((_A,), _f32)

            def ob(i, c):
                ones[i, :] = ov
                return c

            lax.fori_loop(0, _CH, ob, 0)
        plsc.subcore_barrier()

        base = pl.multiple_of(wid * _EPW, _GRP)
        pltpu.async_copy(we_hbm.at[pl.ds(base, _CH), :], webuf[0], wsem[0])

        def grp(g, carry):
            off = pl.multiple_of(base + g * _GRP, _GRP)
            crow = pl.multiple_of(off // _CH, 8)
            pltpu.sync_copy(src.at[pl.ds(crow, _GRP // _CH), :], srcv)
            pltpu.sync_copy(dst.at[pl.ds(crow, _GRP // _CH), :], dstv)
            for j in range(_GRP // _CH):
                b = j % 2
                pltpu.make_async_copy(
                    we_hbm.at[pl.ds(0, _CH), :], webuf[b], wsem[b]).wait()
                if j < _GRP // _CH - 1:
                    noff = pl.multiple_of(off + (j + 1) * _CH, _CH)
                else:
                    noff = pl.multiple_of(
                        jnp.where(g + 1 < _NGRP, off + _GRP, base), _CH)
                pltpu.async_copy(we_hbm.at[pl.ds(noff, _CH), :],
                                 webuf[1 - b], wsem[1 - b])
                pltpu.async_copy(tab_sh.at[srcv.at[j]], xjbuf, sem).wait()
                wb = webuf[b]

                def edge(e, c):
                    xvec = xjbuf[e, :]
                    acc = xvec[0] * wb[e, pl.ds(0, _A)]
                    for i in range(1, _A):
                        acc = acc + xvec[i] * wb[e, pl.ds(i * _A, _A)]
                    mbuf[e, :] = acc
                    return c

                lax.fori_loop(0, _CH, edge, 0, unroll=4)
                pltpu.sync_copy(mbuf, acc_sh.at[dstv.at[j]], add=True)
                if with_deg:
                    pltpu.sync_copy(ones, deg_sh.at[dstv.at[j]], add=True)
            return carry

        lax.fori_loop(0, _NGRP, grp, 0)
        pltpu.make_async_copy(
            we_hbm.at[pl.ds(0, _CH), :], webuf[0], wsem[0]).wait()
        plsc.subcore_barrier()

        pltpu.sync_copy(acc_sh.at[rows, :], zbuf)
        pltpu.sync_copy(zbuf, acc_out.at[cid, rows, :])
        if with_deg:
            pltpu.sync_copy(deg_sh.at[rows, :], zbuf)
            pltpu.sync_copy(zbuf, deg_out.at[cid, rows, :])

    return k(table, src2d, dst2d, we)


def _sc_gather(table, idx2d):
    @functools.partial(
        pl.kernel,
        out_type=jax.ShapeDtypeStruct((_EPAD, _A), _f32),
        mesh=_sc_mesh(),
        scratch_types=[
            pltpu.VMEM((_GRP // _CH, _CH), jnp.int32),
            pltpu.VMEM((_GRP, _A), _f32),
            pltpu.VMEM((_RPT, _A), _f32),
            pltpu.VMEM_SHARED((_NROWP, _A), _f32),
            pltpu.SemaphoreType.DMA,
        ],
        compiler_params=_SC_PARAMS,
    )
    def k(tab, idx, out, idxv, rows, stage, tab_sh, sem):
        sid = lax.axis_index("s")
        wid = sid * _NC + lax.axis_index("c")
        base = wid * _EPW

        trows = pl.ds(pl.multiple_of(sid * _RPT, 8), _RPT)
        pltpu.sync_copy(tab.at[trows, :], stage)
        pltpu.sync_copy(stage, tab_sh.at[trows, :])
        plsc.subcore_barrier()

        def grp(g, carry):
            off = pl.multiple_of(base + g * _GRP, _GRP)
            pltpu.sync_copy(
                idx.at[pl.ds(pl.multiple_of(off // _CH, 8), _GRP // _CH), :],
                idxv)
            descs = [
                pltpu.async_copy(tab_sh.at[idxv.at[j]],
                                 rows.at[pl.ds(j * _CH, _CH), :], sem)
                for j in range(_GRP // _CH)
            ]
            for d in descs:
                d.wait()
            pltpu.sync_copy(rows, out.at[pl.ds(off, _GRP), :])
            return carry

        lax.fori_loop(0, _NGRP, grp, 0)

    return k(table, idx2d)


def _zero_rows(buf, nrows):
    zv = jnp.zeros((_A,), _f32)

    def zb(i, c):
        buf[i, :] = zv
        return c

    lax.fori_loop(0, nrows, zb, 0)


def _sc_scatter(msg, idx2d, with_deg):
    acc_t = jax.ShapeDtypeStruct((_NC, _NROWP, _A), _f32)
    out_type = (acc_t, acc_t) if with_deg else acc_t
    scratch = [
        pltpu.VMEM((_GRP // _CH, _CH), jnp.int32),
        pltpu.VMEM((_GRP, _A), _f32),
        pltpu.VMEM((_RPT, _A), _f32),
        pltpu.VMEM_SHARED((_NROWP, _A), _f32),
    ]
    if with_deg:
        scratch += [
            pltpu.VMEM((_CH, _A), _f32),
            pltpu.VMEM_SHARED((_NROWP, _A), _f32),
        ]

    @functools.partial(pl.kernel, out_type=out_type, mesh=_sc_mesh(),
                       scratch_types=scratch, compiler_params=_SC_PARAMS)
    def k(msg_hbm, idx_hbm, *rest):
        if with_deg:
            acc_out, deg_out, idxv, mbuf, zbuf, acc_sh, ones, deg_sh = rest
        else:
            acc_out, idxv, mbuf, zbuf, acc_sh = rest
            deg_out = ones = deg_sh = None
        cid = lax.axis_index("c")
        sid = lax.axis_index("s")
        wid = sid * _NC + cid
        rows = pl.ds(pl.multiple_of(sid * _RPT, 8), _RPT)

        _zero_rows(zbuf, _RPT)
        pltpu.sync_copy(zbuf, acc_sh.at[rows, :])
        if with_deg:
            pltpu.sync_copy(zbuf, deg_sh.at[rows, :])
            ov = jnp.ones((_A,), _f32)

            def ob(i, c):
                ones[i, :] = ov
                return c

            lax.fori_loop(0, _CH, ob, 0)
        plsc.subcore_barrier()

        def grp(g, carry):
            off = pl.multiple_of(wid * _EPW + g * _GRP, _GRP)
            pltpu.sync_copy(
                idx_hbm.at[pl.ds(pl.multiple_of(off // _CH, 8),
                                 _GRP // _CH), :], idxv)
            pltpu.sync_copy(msg_hbm.at[pl.ds(off, _GRP), :], mbuf)
            for j in range(_GRP // _CH):
                pltpu.sync_copy(mbuf.at[pl.ds(j * _CH, _CH), :],
                                acc_sh.at[idxv.at[j]], add=True)
                if with_deg:
                    pltpu.sync_copy(ones, deg_sh.at[idxv.at[j]], add=True)
            return carry

        lax.fori_loop(0, _NGRP, grp, 0)
        plsc.subcore_barrier()

        pltpu.sync_copy(acc_sh.at[rows, :], zbuf)
        pltpu.sync_copy(zbuf, acc_out.at[cid, rows, :])
        if with_deg:
            pltpu.sync_copy(deg_sh.at[rows, :], zbuf)
            pltpu.sync_copy(zbuf, deg_out.at[cid, rows, :])

    return k(msg, idx2d)


def kernel(x, edge_index, edge_attr, batch, W_lin, b_lin, W_e1, b_e1, W_e2,
           b_e2, conv_bias, W_ih, W_hh, b_ih, b_hh, Wi, Wh, bi, bh, W_pred,
           b_pred):
    pad = _EPAD - _E
    src = jnp.concatenate([edge_index[0], jnp.zeros((pad,), jnp.int32)])
    dst = jnp.concatenate([edge_index[1],
                           jnp.full((pad,), _N, jnp.int32)])
    src2d = src.reshape(_EPAD // _CH, _CH)
    dst2d = dst.reshape(_EPAD // _CH, _CH)
    ea = jnp.concatenate([edge_attr, jnp.zeros((pad, _DE), _f32)])

    wih = jnp.stack([W_ih[k * _A:(k + 1) * _A].T for k in range(3)])
    whh = jnp.stack([W_hh[k * _A:(k + 1) * _A].T for k in range(3)])
    bih = b_ih.reshape(3, _A)
    bhh = b_hh.reshape(3, _A)
    wiq = jnp.stack([Wi[k * _A:(k + 1) * _A, :_A].T for k in range(4)])
    wir = jnp.stack([Wi[k * _A:(k + 1) * _A, _A:].T for k in range(4)])
    wh4 = jnp.stack([Wh[k * _A:(k + 1) * _A].T for k in range(4)])
    bih4 = bi.reshape(4, _A) + bh.reshape(4, _A)
    cb = conv_bias.reshape(1, _A)
    b_lin2 = b_lin.reshape(1, _A)
    b_e12 = b_e1.reshape(1, _CD)
    wpq, wpr = W_pred[:_A], W_pred[_A:]
    bp = b_pred.reshape(1, 1)
    batch2d = batch.reshape(_N, 1)

    h = _tc_node_embed(x, W_lin, b_lin2)
    we = _tc_we(ea, W_e1, b_e12, W_e2, b_e2.reshape(1, -1))
    deg = None
    for t in range(_CONV_N):
        if t == 0:
            acc, deg = _sc_conv(h, src2d, dst2d, we, with_deg=True)
        else:
            acc = _sc_conv(h, src2d, dst2d, we, with_deg=False)
        if t < _CONV_N - 1:
            h = _tc_update(acc, deg, h, cb, wih, whh, bih, bhh)
        else:
            pred = _tc_update_final(acc, deg, h, batch2d, cb, wih, whh, bih,
                                    bhh, wiq, wir, wh4, bih4, wpq, wpr, bp)
    return pred.reshape(-1)


# speedup vs baseline: 3.3989x; 1.0339x over previous
import functools

import jax
import jax.numpy as jnp
from jax import lax
from jax.experimental import pallas as pl
from jax.experimental.pallas import tpu as pltpu
from jax.experimental.pallas import tpu_sc as plsc

_N = 10000
_E = 160000
_DN = 128
_DE = 16
_A = 16
_CD = 32
_B = 64
_CONV_N = 3
_STEPS = 3

_NC, _NS = 2, 16
_NW = _NC * _NS
_CH = 128
_GRP = 1024
_EPW = 5 * _GRP
_EPAD = _NW * _EPW
_NGRP = _EPW // _GRP
_NROWP = 10112
_RPT = _NROWP // _NS

_f32 = jnp.float32


def _tc_node_embed(x, w, b):
    def body(x_ref, w_ref, b_ref, o_ref):
        o_ref[:_N, :] = jnp.maximum(
            jnp.dot(x_ref[...], w_ref[...], preferred_element_type=_f32)
            + b_ref[...], 0.0)
    return pl.pallas_call(
        body, out_shape=jax.ShapeDtypeStruct((_NROWP, _A), _f32))(x, w, b)


_BE = 4096


def _tc_we(ea, w_e1, b_e1, w_e2, b_e2):
    def body(ea_ref, w1_ref, b1_ref, w2_ref, b2_ref, o_ref):
        h2 = jnp.maximum(
            jnp.dot(ea_ref[...], w1_ref[...], preferred_element_type=_f32)
            + b1_ref[...], 0.0)
        o_ref[...] = (jnp.dot(h2, w2_ref[...], preferred_element_type=_f32)
                      + b2_ref[...])

    nb = _EPAD // _BE
    return pl.pallas_call(
        body, grid=(nb,),
        in_specs=[
            pl.BlockSpec((_BE, _DE), lambda i: (i, 0)),
            pl.BlockSpec((_DE, _CD), lambda i: (0, 0)),
            pl.BlockSpec((1, _CD), lambda i: (0, 0)),
            pl.BlockSpec((_CD, _A * _A), lambda i: (0, 0)),
            pl.BlockSpec((1, _A * _A), lambda i: (0, 0)),
        ],
        out_specs=pl.BlockSpec((_BE, _A * _A), lambda i: (i, 0)),
        out_shape=jax.ShapeDtypeStruct((_EPAD, _A * _A), _f32),
    )(ea, w_e1, b_e1, w_e2, b_e2)


def _tc_pad_idx(src_r, dst_r):
    nin = _E // _CH

    def body(s_ref, d_ref, so_ref, do_ref):
        so_ref[:nin, :] = s_ref[...]
        do_ref[:nin, :] = d_ref[...]
        so_ref[nin:, :] = jnp.zeros((_EPAD // _CH - nin, _CH), jnp.int32)
        do_ref[nin:, :] = jnp.full((_EPAD // _CH - nin, _CH), _N, jnp.int32)

    t = jax.ShapeDtypeStruct((_EPAD // _CH, _CH), jnp.int32)
    return pl.pallas_call(body, out_shape=(t, t))(src_r, dst_r)


def _tc_msg(xj, we):
    def body(xj_ref, we_ref, o_ref):
        xj_b = xj_ref[...]
        we_b = we_ref[...]
        acc = xj_b[:, 0][:, None] * we_b[:, 0:_A]
        for i in range(1, _A):
            acc = acc + xj_b[:, i][:, None] * we_b[:, i * _A:(i + 1) * _A]
        o_ref[...] = acc

    nb = _EPAD // _BE
    return pl.pallas_call(
        body, grid=(nb,),
        in_specs=[
            pl.BlockSpec((_BE, _A), lambda i: (i, 0)),
            pl.BlockSpec((_BE, _A * _A), lambda i: (i, 0)),
        ],
        out_specs=pl.BlockSpec((_BE, _A), lambda i: (i, 0)),
        out_shape=jax.ShapeDtypeStruct((_EPAD, _A), _f32),
    )(xj, we)


def _gru(m, h, wih, whh, bih, bhh):
    ir = jnp.dot(m, wih[0], preferred_element_type=_f32) + bih[0][None, :]
    iz = jnp.dot(m, wih[1], preferred_element_type=_f32) + bih[1][None, :]
    inn = jnp.dot(m, wih[2], preferred_element_type=_f32) + bih[2][None, :]
    hr = jnp.dot(h, whh[0], preferred_element_type=_f32) + bhh[0][None, :]
    hz = jnp.dot(h, whh[1], preferred_element_type=_f32) + bhh[1][None, :]
    hn = jnp.dot(h, whh[2], preferred_element_type=_f32) + bhh[2][None, :]
    rr = jax.nn.sigmoid(ir + hr)
    zz = jax.nn.sigmoid(iz + hz)
    nn_ = jnp.tanh(inn + rr * hn)
    return (1.0 - zz) * nn_ + zz * h


def _agg_m(acc_ref, deg_ref, cb_ref):
    s = acc_ref[0, :_N, :] + acc_ref[1, :_N, :]
    dg = deg_ref[0, :_N, :] + deg_ref[1, :_N, :]
    return jnp.maximum(s / jnp.maximum(dg, 1.0) + cb_ref[...], 0.0)


def _tc_update(acc, deg, h, conv_b, wih, whh, bih, bhh):
    def body(acc_ref, deg_ref, h_ref, cb_ref, wih_ref, whh_ref, bih_ref,
             bhh_ref, o_ref):
        m = _agg_m(acc_ref, deg_ref, cb_ref)
        o_ref[:_N, :] = _gru(m, h_ref[:_N, :], wih_ref, whh_ref, bih_ref,
                             bhh_ref)
    return pl.pallas_call(
        body, out_shape=jax.ShapeDtypeStruct((_NROWP, _A), _f32))(
            acc, deg, h, conv_b, wih, whh, bih, bhh)


def _tc_update_final(acc, deg, h, batch2d, conv_b, wih, whh, bih, bhh,
                     wiq, wir, wh4, bih4, wpq, wpr, bp):
    def body(acc_ref, deg_ref, h_ref, b_ref, cb_ref, wih_ref, whh_ref,
             bih_ref, bhh_ref, wiq_ref, wir_ref, wh_ref, bih4_ref,
             wpq_ref, wpr_ref, bp_ref, o_ref):
        m = _agg_m(acc_ref, deg_ref, cb_ref)
        xs = _gru(m, h_ref[:_N, :], wih_ref, whh_ref, bih_ref, bhh_ref)

        onehot = (b_ref[...] ==
                  lax.broadcasted_iota(jnp.int32, (_N, _B), 1)).astype(_f32)
        negmask = (onehot - 1.0) * 1e30

        q = jnp.zeros((_B, _A), _f32)
        r = jnp.zeros((_B, _A), _f32)
        h_s = jnp.zeros((_B, _A), _f32)
        c_s = jnp.zeros((_B, _A), _f32)
        for _ in range(_STEPS):
            g = [jnp.dot(q, wiq_ref[k], preferred_element_type=_f32)
                 + jnp.dot(r, wir_ref[k], preferred_element_type=_f32)
                 + jnp.dot(h_s, wh_ref[k], preferred_element_type=_f32)
                 + bih4_ref[k][None, :] for k in range(4)]
            c_s = jax.nn.sigmoid(g[1]) * c_s + jax.nn.sigmoid(g[0]) * jnp.tanh(g[2])
            h_s = jax.nn.sigmoid(g[3]) * jnp.tanh(c_s)
            q = h_s
            e_mat = lax.dot_general(xs, q, (((1,), (1,)), ((), ())),
                                    preferred_element_type=_f32)
            e_msk = e_mat + negmask
            emax = jnp.max(e_msk, axis=0, keepdims=True)
            a = jnp.exp(e_msk - emax) * onehot
            asum = jnp.sum(a, axis=0, keepdims=True)
            a = a / jnp.maximum(asum, 1e-16)
            r = lax.dot_general(a, xs, (((0,), (0,)), ((), ())),
                                preferred_element_type=_f32)
        o_ref[...] = (jnp.dot(q, wpq_ref[...], preferred_element_type=_f32)
                      + jnp.dot(r, wpr_ref[...], preferred_element_type=_f32)
                      + bp_ref[...])
    return pl.pallas_call(
        body, out_shape=jax.ShapeDtypeStruct((_B, 1), _f32))(
            acc, deg, h, batch2d, conv_b, wih, whh, bih, bhh,
            wiq, wir, wh4, bih4, wpq, wpr, bp)


def _sc_mesh():
    return plsc.VectorSubcoreMesh(core_axis_name="c", subcore_axis_name="s",
                                  num_cores=_NC, num_subcores=_NS)


_SC_PARAMS = pltpu.CompilerParams(use_tc_tiling_on_sc=False)


def _sc_conv(table, src2d, dst2d, we, with_deg):
    acc_t = jax.ShapeDtypeStruct((_NC, _NROWP, _A), _f32)
    out_type = (acc_t, acc_t) if with_deg else acc_t
    scratch = [
        pltpu.VMEM((_GRP // _CH, _CH), jnp.int32),
        pltpu.VMEM((_GRP // _CH, _CH), jnp.int32),
        pltpu.VMEM((_CH, _A * _A), _f32),
        pltpu.VMEM((_CH, _A * _A), _f32),
        pltpu.VMEM((_CH, _A), _f32),
        pltpu.VMEM((_CH, _A), _f32),
        pltpu.VMEM((_RPT, _A), _f32),
        pltpu.VMEM_SHARED((_NROWP, _A), _f32),
        pltpu.VMEM_SHARED((_NROWP, _A), _f32),
        pltpu.SemaphoreType.DMA,
        pltpu.SemaphoreType.DMA,
        pltpu.SemaphoreType.DMA,
    ]
    if with_deg:
        scratch += [
            pltpu.VMEM((_CH, _A), _f32),
            pltpu.VMEM_SHARED((_NROWP, _A), _f32),
        ]

    @functools.partial(pl.kernel, out_type=out_type, mesh=_sc_mesh(),
                       scratch_types=scratch, compiler_params=_SC_PARAMS)
    def k(tab, src, dst, we_hbm, *rest):
        if with_deg:
            (acc_out, deg_out, srcv, dstv, web0, web1, xjbuf, mbuf, zbuf,
             tab_sh, acc_sh, sem, wsem0, wsem1, ones, deg_sh) = rest
        else:
            (acc_out, srcv, dstv, web0, web1, xjbuf, mbuf, zbuf,
             tab_sh, acc_sh, sem, wsem0, wsem1) = rest
            deg_out = ones = deg_sh = None
        webuf = (web0, web1)
        wsem = (wsem0, wsem1)
        cid = lax.axis_index("c")
        sid = lax.axis_index("s")
        wid = sid * _NC + cid
        rows = pl.ds(pl.multiple_of(sid * _RPT, 8), _RPT)

        pltpu.sync_copy(tab.at[rows, :], zbuf)
        pltpu.sync_copy(zbuf, tab_sh.at[rows, :])
        _zero_rows(zbuf, _RPT)
        pltpu.sync_copy(zbuf, acc_sh.at[rows, :])
        if with_deg:
            pltpu.sync_copy(zbuf, deg_sh.at[rows, :])
            ov = jnp.ones((_A,), _f32)

            def ob(i, c):
                ones[i, :] = ov
                return c

            lax.fori_loop(0, _CH, ob, 0)
        plsc.subcore_barrier()

        base = pl.multiple_of(wid * _EPW, _GRP)
        pltpu.async_copy(we_hbm.at[pl.ds(base, _CH), :], webuf[0], wsem[0])

        def grp(g, carry):
            off = pl.multiple_of(base + g * _GRP, _GRP)
            crow = pl.multiple_of(off // _CH, 8)
            pltpu.sync_copy(src.at[pl.ds(crow, _GRP // _CH), :], srcv)
            pltpu.sync_copy(dst.at[pl.ds(crow, _GRP // _CH), :], dstv)
            for j in range(_GRP // _CH):
                b = j % 2
                pltpu.make_async_copy(
                    we_hbm.at[pl.ds(0, _CH), :], webuf[b], wsem[b]).wait()
                if j < _GRP // _CH - 1:
                    noff = pl.multiple_of(off + (j + 1) * _CH, _CH)
                else:
                    noff = pl.multiple_of(
                        jnp.where(g + 1 < _NGRP, off + _GRP, base), _CH)
                pltpu.async_copy(we_hbm.at[pl.ds(noff, _CH), :],
                                 webuf[1 - b], wsem[1 - b])
                pltpu.async_copy(tab_sh.at[srcv.at[j]], xjbuf, sem).wait()
                wb = webuf[b]

                def edge(e, c):
                    xvec = xjbuf[e, :]
                    acc = xvec[0] * wb[e, pl.ds(0, _A)]
                    for i in range(1, _A):
                        acc = acc + xvec[i] * wb[e, pl.ds(i * _A, _A)]
                    mbuf[e, :] = acc
                    return c

                lax.fori_loop(0, _CH, edge, 0, unroll=4)
                pltpu.sync_copy(mbuf, acc_sh.at[dstv.at[j]], add=True)
                if with_deg:
                    pltpu.sync_copy(ones, deg_sh.at[dstv.at[j]], add=True)
            return carry

        lax.fori_loop(0, _NGRP, grp, 0)
        pltpu.make_async_copy(
            we_hbm.at[pl.ds(0, _CH), :], webuf[0], wsem[0]).wait()
        plsc.subcore_barrier()

        pltpu.sync_copy(acc_sh.at[rows, :], zbuf)
        pltpu.sync_copy(zbuf, acc_out.at[cid, rows, :])
        if with_deg:
            pltpu.sync_copy(deg_sh.at[rows, :], zbuf)
            pltpu.sync_copy(zbuf, deg_out.at[cid, rows, :])

    return k(table, src2d, dst2d, we)


def _sc_gather(table, idx2d):
    @functools.partial(
        pl.kernel,
        out_type=jax.ShapeDtypeStruct((_EPAD, _A), _f32),
        mesh=_sc_mesh(),
        scratch_types=[
            pltpu.VMEM((_GRP // _CH, _CH), jnp.int32),
            pltpu.VMEM((_GRP, _A), _f32),
            pltpu.VMEM((_RPT, _A), _f32),
            pltpu.VMEM_SHARED((_NROWP, _A), _f32),
            pltpu.SemaphoreType.DMA,
        ],
        compiler_params=_SC_PARAMS,
    )
    def k(tab, idx, out, idxv, rows, stage, tab_sh, sem):
        sid = lax.axis_index("s")
        wid = sid * _NC + lax.axis_index("c")
        base = wid * _EPW

        trows = pl.ds(pl.multiple_of(sid * _RPT, 8), _RPT)
        pltpu.sync_copy(tab.at[trows, :], stage)
        pltpu.sync_copy(stage, tab_sh.at[trows, :])
        plsc.subcore_barrier()

        def grp(g, carry):
            off = pl.multiple_of(base + g * _GRP, _GRP)
            pltpu.sync_copy(
                idx.at[pl.ds(pl.multiple_of(off // _CH, 8), _GRP // _CH), :],
                idxv)
            descs = [
                pltpu.async_copy(tab_sh.at[idxv.at[j]],
                                 rows.at[pl.ds(j * _CH, _CH), :], sem)
                for j in range(_GRP // _CH)
            ]
            for d in descs:
                d.wait()
            pltpu.sync_copy(rows, out.at[pl.ds(off, _GRP), :])
            return carry

        lax.fori_loop(0, _NGRP, grp, 0)

    return k(table, idx2d)


def _zero_rows(buf, nrows):
    zv = jnp.zeros((_A,), _f32)

    def zb(i, c):
        buf[i, :] = zv
        return c

    lax.fori_loop(0, nrows, zb, 0)


def _sc_scatter(msg, idx2d, with_deg):
    acc_t = jax.ShapeDtypeStruct((_NC, _NROWP, _A), _f32)
    out_type = (acc_t, acc_t) if with_deg else acc_t
    scratch = [
        pltpu.VMEM((_GRP // _CH, _CH), jnp.int32),
        pltpu.VMEM((_GRP, _A), _f32),
        pltpu.VMEM((_RPT, _A), _f32),
        pltpu.VMEM_SHARED((_NROWP, _A), _f32),
    ]
    if with_deg:
        scratch += [
            pltpu.VMEM((_CH, _A), _f32),
            pltpu.VMEM_SHARED((_NROWP, _A), _f32),
        ]

    @functools.partial(pl.kernel, out_type=out_type, mesh=_sc_mesh(),
                       scratch_types=scratch, compiler_params=_SC_PARAMS)
    def k(msg_hbm, idx_hbm, *rest):
        if with_deg:
            acc_out, deg_out, idxv, mbuf, zbuf, acc_sh, ones, deg_sh = rest
        else:
            acc_out, idxv, mbuf, zbuf, acc_sh = rest
            deg_out = ones = deg_sh = None
        cid = lax.axis_index("c")
        sid = lax.axis_index("s")
        wid = sid * _NC + cid
        rows = pl.ds(pl.multiple_of(sid * _RPT, 8), _RPT)

        _zero_rows(zbuf, _RPT)
        pltpu.sync_copy(zbuf, acc_sh.at[rows, :])
        if with_deg:
            pltpu.sync_copy(zbuf, deg_sh.at[rows, :])
            ov = jnp.ones((_A,), _f32)

            def ob(i, c):
                ones[i, :] = ov
                return c

            lax.fori_loop(0, _CH, ob, 0)
        plsc.subcore_barrier()

        def grp(g, carry):
            off = pl.multiple_of(wid * _EPW + g * _GRP, _GRP)
            pltpu.sync_copy(
                idx_hbm.at[pl.ds(pl.multiple_of(off // _CH, 8),
                                 _GRP // _CH), :], idxv)
            pltpu.sync_copy(msg_hbm.at[pl.ds(off, _GRP), :], mbuf)
            for j in range(_GRP // _CH):
                pltpu.sync_copy(mbuf.at[pl.ds(j * _CH, _CH), :],
                                acc_sh.at[idxv.at[j]], add=True)
                if with_deg:
                    pltpu.sync_copy(ones, deg_sh.at[idxv.at[j]], add=True)
            return carry

        lax.fori_loop(0, _NGRP, grp, 0)
        plsc.subcore_barrier()

        pltpu.sync_copy(acc_sh.at[rows, :], zbuf)
        pltpu.sync_copy(zbuf, acc_out.at[cid, rows, :])
        if with_deg:
            pltpu.sync_copy(deg_sh.at[rows, :], zbuf)
            pltpu.sync_copy(zbuf, deg_out.at[cid, rows, :])

    return k(msg, idx2d)


def kernel(x, edge_index, edge_attr, batch, W_lin, b_lin, W_e1, b_e1, W_e2,
           b_e2, conv_bias, W_ih, W_hh, b_ih, b_hh, Wi, Wh, bi, bh, W_pred,
           b_pred):
    src2d, dst2d = _tc_pad_idx(edge_index[0].reshape(_E // _CH, _CH),
                               edge_index[1].reshape(_E // _CH, _CH))
    ea = edge_attr

    wih = jnp.stack([W_ih[k * _A:(k + 1) * _A].T for k in range(3)])
    whh = jnp.stack([W_hh[k * _A:(k + 1) * _A].T for k in range(3)])
    bih = b_ih.reshape(3, _A)
    bhh = b_hh.reshape(3, _A)
    wiq = jnp.stack([Wi[k * _A:(k + 1) * _A, :_A].T for k in range(4)])
    wir = jnp.stack([Wi[k * _A:(k + 1) * _A, _A:].T for k in range(4)])
    wh4 = jnp.stack([Wh[k * _A:(k + 1) * _A].T for k in range(4)])
    bih4 = bi.reshape(4, _A) + bh.reshape(4, _A)
    cb = conv_bias.reshape(1, _A)
    b_lin2 = b_lin.reshape(1, _A)
    b_e12 = b_e1.reshape(1, _CD)
    wpq, wpr = W_pred[:_A], W_pred[_A:]
    bp = b_pred.reshape(1, 1)
    batch2d = batch.reshape(_N, 1)

    h = _tc_node_embed(x, W_lin, b_lin2)
    we = _tc_we(ea, W_e1, b_e12, W_e2, b_e2.reshape(1, -1))
    deg = None
    for t in range(_CONV_N):
        if t == 0:
            acc, deg = _sc_conv(h, src2d, dst2d, we, with_deg=True)
        else:
            acc = _sc_conv(h, src2d, dst2d, we, with_deg=False)
        if t < _CONV_N - 1:
            h = _tc_update(acc, deg, h, cb, wih, whh, bih, bhh)
        else:
            pred = _tc_update_final(acc, deg, h, batch2d, cb, wih, whh, bih,
                                    bhh, wiq, wir, wh4, bih4, wpq, wpr, bp)
    return pred.reshape(-1)


# speedup vs baseline: 3.9115x; 1.1508x over previous
import functools

import jax
import jax.numpy as jnp
from jax import lax
from jax.experimental import pallas as pl
from jax.experimental.pallas import tpu as pltpu
from jax.experimental.pallas import tpu_sc as plsc

_N = 10000
_E = 160000
_DN = 128
_DE = 16
_A = 16
_CD = 32
_B = 64
_CONV_N = 3
_STEPS = 3

_NC, _NS = 2, 16
_NW = _NC * _NS
_CH = 128
_GRP = 1024
_EPW = 5 * _GRP
_EPAD = _NW * _EPW
_NGRP = _EPW // _GRP
_NROWP = 10112
_RPT = _NROWP // _NS

_f32 = jnp.float32


def _tc_node_embed(x, w, b):
    def body(x_ref, w_ref, b_ref, o_ref):
        o_ref[:_N, :] = jnp.maximum(
            jnp.dot(x_ref[...], w_ref[...], preferred_element_type=_f32)
            + b_ref[...], 0.0)
    return pl.pallas_call(
        body, out_shape=jax.ShapeDtypeStruct((_NROWP, _A), _f32))(x, w, b)


_BE = 4096


def _tc_we(ea, w_e1, b_e1, w_e2, b_e2):
    def body(ea_ref, w1_ref, b1_ref, w2_ref, b2_ref, lo_ref, hi_ref):
        h2 = jnp.maximum(
            jnp.dot(ea_ref[...], w1_ref[...], preferred_element_type=_f32)
            + b1_ref[...], 0.0)
        we_full = (jnp.dot(h2, w2_ref[...], preferred_element_type=_f32)
                   + b2_ref[...])
        lo_ref[...] = we_full[:, :_CH]
        hi_ref[...] = we_full[:, _CH:]

    nb = _EPAD // _BE
    return pl.pallas_call(
        body, grid=(nb,),
        in_specs=[
            pl.BlockSpec((_BE, _DE), lambda i: (i, 0)),
            pl.BlockSpec((_DE, _CD), lambda i: (0, 0)),
            pl.BlockSpec((1, _CD), lambda i: (0, 0)),
            pl.BlockSpec((_CD, _A * _A), lambda i: (0, 0)),
            pl.BlockSpec((1, _A * _A), lambda i: (0, 0)),
        ],
        out_specs=(pl.BlockSpec((_BE, _CH), lambda i: (i, 0)),
                   pl.BlockSpec((_BE, _CH), lambda i: (i, 0))),
        out_shape=(jax.ShapeDtypeStruct((_EPAD, _CH), _f32),
                   jax.ShapeDtypeStruct((_EPAD, _CH), _f32)),
    )(ea, w_e1, b_e1, w_e2, b_e2)


def _tc_pad_idx(src_r, dst_r):
    nin = _E // _CH

    def body(s_ref, d_ref, so_ref, do_ref):
        so_ref[:nin, :] = s_ref[...]
        do_ref[:nin, :] = d_ref[...]
        so_ref[nin:, :] = jnp.zeros((_EPAD // _CH - nin, _CH), jnp.int32)
        do_ref[nin:, :] = jnp.full((_EPAD // _CH - nin, _CH), _N, jnp.int32)

    t = jax.ShapeDtypeStruct((_EPAD // _CH, _CH), jnp.int32)
    return pl.pallas_call(body, out_shape=(t, t))(src_r, dst_r)


def _tc_msg(xj, we):
    def body(xj_ref, we_ref, o_ref):
        xj_b = xj_ref[...]
        we_b = we_ref[...]
        acc = xj_b[:, 0][:, None] * we_b[:, 0:_A]
        for i in range(1, _A):
            acc = acc + xj_b[:, i][:, None] * we_b[:, i * _A:(i + 1) * _A]
        o_ref[...] = acc

    nb = _EPAD // _BE
    return pl.pallas_call(
        body, grid=(nb,),
        in_specs=[
            pl.BlockSpec((_BE, _A), lambda i: (i, 0)),
            pl.BlockSpec((_BE, _A * _A), lambda i: (i, 0)),
        ],
        out_specs=pl.BlockSpec((_BE, _A), lambda i: (i, 0)),
        out_shape=jax.ShapeDtypeStruct((_EPAD, _A), _f32),
    )(xj, we)


def _gru(m, h, wih, whh, bih, bhh):
    ir = jnp.dot(m, wih[0], preferred_element_type=_f32) + bih[0][None, :]
    iz = jnp.dot(m, wih[1], preferred_element_type=_f32) + bih[1][None, :]
    inn = jnp.dot(m, wih[2], preferred_element_type=_f32) + bih[2][None, :]
    hr = jnp.dot(h, whh[0], preferred_element_type=_f32) + bhh[0][None, :]
    hz = jnp.dot(h, whh[1], preferred_element_type=_f32) + bhh[1][None, :]
    hn = jnp.dot(h, whh[2], preferred_element_type=_f32) + bhh[2][None, :]
    rr = jax.nn.sigmoid(ir + hr)
    zz = jax.nn.sigmoid(iz + hz)
    nn_ = jnp.tanh(inn + rr * hn)
    return (1.0 - zz) * nn_ + zz * h


def _agg_m(acc_ref, deg_ref, cb_ref):
    s = acc_ref[0, :_N, :] + acc_ref[1, :_N, :]
    dg = deg_ref[0, :_N, :] + deg_ref[1, :_N, :]
    return jnp.maximum(s / jnp.maximum(dg, 1.0) + cb_ref[...], 0.0)


def _tc_update(acc, deg, h, conv_b, wih, whh, bih, bhh):
    def body(acc_ref, deg_ref, h_ref, cb_ref, wih_ref, whh_ref, bih_ref,
             bhh_ref, o_ref):
        m = _agg_m(acc_ref, deg_ref, cb_ref)
        o_ref[:_N, :] = _gru(m, h_ref[:_N, :], wih_ref, whh_ref, bih_ref,
                             bhh_ref)
    return pl.pallas_call(
        body, out_shape=jax.ShapeDtypeStruct((_NROWP, _A), _f32))(
            acc, deg, h, conv_b, wih, whh, bih, bhh)


def _tc_update_final(acc, deg, h, batch2d, conv_b, wih, whh, bih, bhh,
                     wiq, wir, wh4, bih4, wpq, wpr, bp):
    def body(acc_ref, deg_ref, h_ref, b_ref, cb_ref, wih_ref, whh_ref,
             bih_ref, bhh_ref, wiq_ref, wir_ref, wh_ref, bih4_ref,
             wpq_ref, wpr_ref, bp_ref, o_ref):
        m = _agg_m(acc_ref, deg_ref, cb_ref)
        xs = _gru(m, h_ref[:_N, :], wih_ref, whh_ref, bih_ref, bhh_ref)

        onehot = (b_ref[...] ==
                  lax.broadcasted_iota(jnp.int32, (_N, _B), 1)).astype(_f32)
        negmask = (onehot - 1.0) * 1e30

        q = jnp.zeros((_B, _A), _f32)
        r = jnp.zeros((_B, _A), _f32)
        h_s = jnp.zeros((_B, _A), _f32)
        c_s = jnp.zeros((_B, _A), _f32)
        for _ in range(_STEPS):
            g = [jnp.dot(q, wiq_ref[k], preferred_element_type=_f32)
                 + jnp.dot(r, wir_ref[k], preferred_element_type=_f32)
                 + jnp.dot(h_s, wh_ref[k], preferred_element_type=_f32)
                 + bih4_ref[k][None, :] for k in range(4)]
            c_s = jax.nn.sigmoid(g[1]) * c_s + jax.nn.sigmoid(g[0]) * jnp.tanh(g[2])
            h_s = jax.nn.sigmoid(g[3]) * jnp.tanh(c_s)
            q = h_s
            e_mat = lax.dot_general(xs, q, (((1,), (1,)), ((), ())),
                                    preferred_element_type=_f32)
            e_msk = e_mat + negmask
            emax = jnp.max(e_msk, axis=0, keepdims=True)
            a = jnp.exp(e_msk - emax) * onehot
            asum = jnp.sum(a, axis=0, keepdims=True)
            a = a / jnp.maximum(asum, 1e-16)
            r = lax.dot_general(a, xs, (((0,), (0,)), ((), ())),
                                preferred_element_type=_f32)
        o_ref[...] = (jnp.dot(q, wpq_ref[...], preferred_element_type=_f32)
                      + jnp.dot(r, wpr_ref[...], preferred_element_type=_f32)
                      + bp_ref[...])
    return pl.pallas_call(
        body, out_shape=jax.ShapeDtypeStruct((_B, 1), _f32))(
            acc, deg, h, batch2d, conv_b, wih, whh, bih, bhh,
            wiq, wir, wh4, bih4, wpq, wpr, bp)


def _sc_mesh():
    return plsc.VectorSubcoreMesh(core_axis_name="c", subcore_axis_name="s",
                                  num_cores=_NC, num_subcores=_NS)


_SC_PARAMS = pltpu.CompilerParams(use_tc_tiling_on_sc=False)


def _sc_conv(table, src2d, dst2d, we_lo, we_hi, with_deg):
    acc_t = jax.ShapeDtypeStruct((_NC, _NROWP, _A), _f32)
    out_type = (acc_t, acc_t) if with_deg else acc_t
    scratch = [
        pltpu.VMEM((_GRP // _CH, _CH), jnp.int32),
        pltpu.VMEM((_GRP // _CH, _CH), jnp.int32),
        pltpu.VMEM((_CH, _CH), _f32),
        pltpu.VMEM((_CH, _CH), _f32),
        pltpu.VMEM((_CH, _CH), _f32),
        pltpu.VMEM((_CH, _CH), _f32),
        pltpu.VMEM((_CH, _A), _f32),
        pltpu.VMEM((_CH, _A), _f32),
        pltpu.VMEM((_RPT, _A), _f32),
        pltpu.VMEM_SHARED((_NROWP, _A), _f32),
        pltpu.VMEM_SHARED((_NROWP, _A), _f32),
        pltpu.SemaphoreType.DMA,
        pltpu.SemaphoreType.DMA,
        pltpu.SemaphoreType.DMA,
    ]
    if with_deg:
        scratch += [
            pltpu.VMEM((_CH, _A), _f32),
            pltpu.VMEM_SHARED((_NROWP, _A), _f32),
        ]

    @functools.partial(pl.kernel, out_type=out_type, mesh=_sc_mesh(),
                       scratch_types=scratch, compiler_params=_SC_PARAMS)
    def k(tab, src, dst, welo_hbm, wehi_hbm, *rest):
        if with_deg:
            (acc_out, deg_out, srcv, dstv, wlo0, whi0, wlo1, whi1, xjbuf,
             mbuf, zbuf, tab_sh, acc_sh, sem, wsem0, wsem1, ones,
             deg_sh) = rest
        else:
            (acc_out, srcv, dstv, wlo0, whi0, wlo1, whi1, xjbuf, mbuf,
             zbuf, tab_sh, acc_sh, sem, wsem0, wsem1) = rest
            deg_out = ones = deg_sh = None
        webuf = ((wlo0, whi0), (wlo1, whi1))
        wsem = (wsem0, wsem1)
        cid = lax.axis_index("c")
        sid = lax.axis_index("s")
        wid = sid * _NC + cid
        rows = pl.ds(pl.multiple_of(sid * _RPT, 8), _RPT)

        pltpu.sync_copy(tab.at[rows, :], zbuf)
        pltpu.sync_copy(zbuf, tab_sh.at[rows, :])
        _zero_rows(zbuf, _RPT)
        pltpu.sync_copy(zbuf, acc_sh.at[rows, :])
        if with_deg:
            pltpu.sync_copy(zbuf, deg_sh.at[rows, :])
            ov = jnp.ones((_A,), _f32)

            def ob(i, c):
                ones[i, :] = ov
                return c

            lax.fori_loop(0, _CH, ob, 0)
        plsc.subcore_barrier()

        base = pl.multiple_of(wid * _EPW, _GRP)
        pltpu.async_copy(welo_hbm.at[pl.ds(base, _CH), :], webuf[0][0],
                         wsem[0])
        pltpu.async_copy(wehi_hbm.at[pl.ds(base, _CH), :], webuf[0][1],
                         wsem[0])

        def grp(g, carry):
            off = pl.multiple_of(base + g * _GRP, _GRP)
            crow = pl.multiple_of(off // _CH, 8)
            pltpu.sync_copy(src.at[pl.ds(crow, _GRP // _CH), :], srcv)
            pltpu.sync_copy(dst.at[pl.ds(crow, _GRP // _CH), :], dstv)
            for j in range(_GRP // _CH):
                b = j % 2
                pltpu.make_async_copy(
                    welo_hbm.at[pl.ds(0, _CH), :], webuf[b][0],
                    wsem[b]).wait()
                pltpu.make_async_copy(
                    wehi_hbm.at[pl.ds(0, _CH), :], webuf[b][1],
                    wsem[b]).wait()
                if j < _GRP // _CH - 1:
                    noff = pl.multiple_of(off + (j + 1) * _CH, _CH)
                else:
                    noff = pl.multiple_of(
                        jnp.where(g + 1 < _NGRP, off + _GRP, base), _CH)
                pltpu.async_copy(welo_hbm.at[pl.ds(noff, _CH), :],
                                 webuf[1 - b][0], wsem[1 - b])
                pltpu.async_copy(wehi_hbm.at[pl.ds(noff, _CH), :],
                                 webuf[1 - b][1], wsem[1 - b])
                pltpu.async_copy(tab_sh.at[srcv.at[j]], xjbuf, sem).wait()
                wlo, whi = webuf[b]

                def edge(e, c):
                    xvec = xjbuf[e, :]
                    acc = xvec[0] * wlo[e, pl.ds(0, _A)]
                    for i in range(1, 8):
                        acc = acc + xvec[i] * wlo[e, pl.ds(i * _A, _A)]
                    for i in range(8, _A):
                        acc = acc + xvec[i] * whi[e, pl.ds((i - 8) * _A, _A)]
                    mbuf[e, :] = acc
                    return c

                lax.fori_loop(0, _CH, edge, 0, unroll=4)
                pltpu.sync_copy(mbuf, acc_sh.at[dstv.at[j]], add=True)
                if with_deg:
                    pltpu.sync_copy(ones, deg_sh.at[dstv.at[j]], add=True)
            return carry

        lax.fori_loop(0, _NGRP, grp, 0)
        pltpu.make_async_copy(
            welo_hbm.at[pl.ds(0, _CH), :], webuf[0][0], wsem[0]).wait()
        pltpu.make_async_copy(
            wehi_hbm.at[pl.ds(0, _CH), :], webuf[0][1], wsem[0]).wait()
        plsc.subcore_barrier()

        pltpu.sync_copy(acc_sh.at[rows, :], zbuf)
        pltpu.sync_copy(zbuf, acc_out.at[cid, rows, :])
        if with_deg:
            pltpu.sync_copy(deg_sh.at[rows, :], zbuf)
            pltpu.sync_copy(zbuf, deg_out.at[cid, rows, :])

    return k(table, src2d, dst2d, we_lo, we_hi)


def _sc_gather(table, idx2d):
    @functools.partial(
        pl.kernel,
        out_type=jax.ShapeDtypeStruct((_EPAD, _A), _f32),
        mesh=_sc_mesh(),
        scratch_types=[
            pltpu.VMEM((_GRP // _CH, _CH), jnp.int32),
            pltpu.VMEM((_GRP, _A), _f32),
            pltpu.VMEM((_RPT, _A), _f32),
            pltpu.VMEM_SHARED((_NROWP, _A), _f32),
            pltpu.SemaphoreType.DMA,
        ],
        compiler_params=_SC_PARAMS,
    )
    def k(tab, idx, out, idxv, rows, stage, tab_sh, sem):
        sid = lax.axis_index("s")
        wid = sid * _NC + lax.axis_index("c")
        base = wid * _EPW

        trows = pl.ds(pl.multiple_of(sid * _RPT, 8), _RPT)
        pltpu.sync_copy(tab.at[trows, :], stage)
        pltpu.sync_copy(stage, tab_sh.at[trows, :])
        plsc.subcore_barrier()

        def grp(g, carry):
            off = pl.multiple_of(base + g * _GRP, _GRP)
            pltpu.sync_copy(
                idx.at[pl.ds(pl.multiple_of(off // _CH, 8), _GRP // _CH), :],
                idxv)
            descs = [
                pltpu.async_copy(tab_sh.at[idxv.at[j]],
                                 rows.at[pl.ds(j * _CH, _CH), :], sem)
                for j in range(_GRP // _CH)
            ]
            for d in descs:
                d.wait()
            pltpu.sync_copy(rows, out.at[pl.ds(off, _GRP), :])
            return carry

        lax.fori_loop(0, _NGRP, grp, 0)

    return k(table, idx2d)


def _zero_rows(buf, nrows):
    zv = jnp.zeros((_A,), _f32)

    def zb(i, c):
        buf[i, :] = zv
        return c

    lax.fori_loop(0, nrows, zb, 0)


def _sc_scatter(msg, idx2d, with_deg):
    acc_t = jax.ShapeDtypeStruct((_NC, _NROWP, _A), _f32)
    out_type = (acc_t, acc_t) if with_deg else acc_t
    scratch = [
        pltpu.VMEM((_GRP // _CH, _CH), jnp.int32),
        pltpu.VMEM((_GRP, _A), _f32),
        pltpu.VMEM((_RPT, _A), _f32),
        pltpu.VMEM_SHARED((_NROWP, _A), _f32),
    ]
    if with_deg:
        scratch += [
            pltpu.VMEM((_CH, _A), _f32),
            pltpu.VMEM_SHARED((_NROWP, _A), _f32),
        ]

    @functools.partial(pl.kernel, out_type=out_type, mesh=_sc_mesh(),
                       scratch_types=scratch, compiler_params=_SC_PARAMS)
    def k(msg_hbm, idx_hbm, *rest):
        if with_deg:
            acc_out, deg_out, idxv, mbuf, zbuf, acc_sh, ones, deg_sh = rest
        else:
            acc_out, idxv, mbuf, zbuf, acc_sh = rest
            deg_out = ones = deg_sh = None
        cid = lax.axis_index("c")
        sid = lax.axis_index("s")
        wid = sid * _NC + cid
        rows = pl.ds(pl.multiple_of(sid * _RPT, 8), _RPT)

        _zero_rows(zbuf, _RPT)
        pltpu.sync_copy(zbuf, acc_sh.at[rows, :])
        if with_deg:
            pltpu.sync_copy(zbuf, deg_sh.at[rows, :])
            ov = jnp.ones((_A,), _f32)

            def ob(i, c):
                ones[i, :] = ov
                return c

            lax.fori_loop(0, _CH, ob, 0)
        plsc.subcore_barrier()

        def grp(g, carry):
            off = pl.multiple_of(wid * _EPW + g * _GRP, _GRP)
            pltpu.sync_copy(
                idx_hbm.at[pl.ds(pl.multiple_of(off // _CH, 8),
                                 _GRP // _CH), :], idxv)
            pltpu.sync_copy(msg_hbm.at[pl.ds(off, _GRP), :], mbuf)
            for j in range(_GRP // _CH):
                pltpu.sync_copy(mbuf.at[pl.ds(j * _CH, _CH), :],
                                acc_sh.at[idxv.at[j]], add=True)
                if with_deg:
                    pltpu.sync_copy(ones, deg_sh.at[idxv.at[j]], add=True)
            return carry

        lax.fori_loop(0, _NGRP, grp, 0)
        plsc.subcore_barrier()

        pltpu.sync_copy(acc_sh.at[rows, :], zbuf)
        pltpu.sync_copy(zbuf, acc_out.at[cid, rows, :])
        if with_deg:
            pltpu.sync_copy(deg_sh.at[rows, :], zbuf)
            pltpu.sync_copy(zbuf, deg_out.at[cid, rows, :])

    return k(msg, idx2d)


def kernel(x, edge_index, edge_attr, batch, W_lin, b_lin, W_e1, b_e1, W_e2,
           b_e2, conv_bias, W_ih, W_hh, b_ih, b_hh, Wi, Wh, bi, bh, W_pred,
           b_pred):
    src2d, dst2d = _tc_pad_idx(edge_index[0].reshape(_E // _CH, _CH),
                               edge_index[1].reshape(_E // _CH, _CH))
    ea = edge_attr

    wih = jnp.stack([W_ih[k * _A:(k + 1) * _A].T for k in range(3)])
    whh = jnp.stack([W_hh[k * _A:(k + 1) * _A].T for k in range(3)])
    bih = b_ih.reshape(3, _A)
    bhh = b_hh.reshape(3, _A)
    wiq = jnp.stack([Wi[k * _A:(k + 1) * _A, :_A].T for k in range(4)])
    wir = jnp.stack([Wi[k * _A:(k + 1) * _A, _A:].T for k in range(4)])
    wh4 = jnp.stack([Wh[k * _A:(k + 1) * _A].T for k in range(4)])
    bih4 = bi.reshape(4, _A) + bh.reshape(4, _A)
    cb = conv_bias.reshape(1, _A)
    b_lin2 = b_lin.reshape(1, _A)
    b_e12 = b_e1.reshape(1, _CD)
    wpq, wpr = W_pred[:_A], W_pred[_A:]
    bp = b_pred.reshape(1, 1)
    batch2d = batch.reshape(_N, 1)

    h = _tc_node_embed(x, W_lin, b_lin2)
    we_lo, we_hi = _tc_we(ea, W_e1, b_e12, W_e2, b_e2.reshape(1, -1))
    deg = None
    for t in range(_CONV_N):
        if t == 0:
            acc, deg = _sc_conv(h, src2d, dst2d, we_lo, we_hi,
                                with_deg=True)
        else:
            acc = _sc_conv(h, src2d, dst2d, we_lo, we_hi, with_deg=False)
        if t < _CONV_N - 1:
            h = _tc_update(acc, deg, h, cb, wih, whh, bih, bhh)
        else:
            pred = _tc_update_final(acc, deg, h, batch2d, cb, wih, whh, bih,
                                    bhh, wiq, wir, wh4, bih4, wpq, wpr, bp)
    return pred.reshape(-1)
